# baseline (speedup 1.0000x reference)
import jax
import jax.numpy as jnp
from jax.experimental import pallas as pl
from jax.experimental.pallas import tpu as pltpu

_BL = 4096
_K = 304


def _reduce_body(eo_ref, el_ref, en_ref, cp_ref):
    x = eo_ref[...]
    ss = jnp.sum(x * x, axis=1, keepdims=True)
    en_ref[0] = jnp.sqrt(ss).reshape(_BL // 256, 256)
    ml = jnp.max(el_ref[...], axis=1, keepdims=True)
    cp_ref[0] = jax.nn.sigmoid(ml).reshape(_BL // 256, 256)


def _reduce_pass(eo2, el2, B, L):
    n = B * L
    nb = n // _BL
    rows = _BL // 256
    bpb = L // _BL
    return pl.pallas_call(
        _reduce_body,
        grid=(nb,),
        in_specs=[
            pl.BlockSpec((_BL, 256), lambda i: (i, 0)),
            pl.BlockSpec((_BL, 80), lambda i: (i, 0)),
        ],
        out_specs=[
            pl.BlockSpec((1, rows, 256), lambda i: (i // bpb, i % bpb, 0)),
            pl.BlockSpec((1, rows, 256), lambda i: (i // bpb, i % bpb, 0)),
        ],
        out_shape=[
            jax.ShapeDtypeStruct((B, 256, 256), jnp.float32),
            jax.ShapeDtypeStruct((B, 256, 256), jnp.float32),
        ],
    )(eo2, el2)


def _score_one(e, cp):
    zr = jnp.zeros((1, 256), jnp.float32)
    zc = jnp.zeros((256, 1), jnp.float32)

    def sh(a, dh, dw):
        if dh == 1:
            a = jnp.concatenate([a[1:, :], zr], axis=0)
        elif dh == -1:
            a = jnp.concatenate([zr, a[:-1, :]], axis=0)
        if dw == 1:
            a = jnp.concatenate([a[:, 1:], zc], axis=1)
        elif dw == -1:
            a = jnp.concatenate([zc, a[:, :-1]], axis=1)
        return a

    win = sh(e, -1, -1)
    for dh, dw in ((-1, 0), (-1, 1), (0, -1), (0, 0), (0, 1),
                   (1, -1), (1, 0), (1, 1)):
        win = win + sh(e, dh, dw)
    dens = win * jnp.float32(1.0 / 9.0)
    mn = jnp.min(dens)
    mx = jnp.max(dens)
    denom = (mx - mn) + 1e-06
    return cp * (1.0 - 0.4) + ((dens - mn) / denom) * 0.4


def _rm_cumsum(m):
    x = m
    for sh in (1, 2, 4, 8, 16, 32, 64, 128):
        x = x + jnp.concatenate(
            [jnp.zeros((256, sh), jnp.int32), x[:, :256 - sh]], axis=1)
    e = jnp.concatenate(
        [jnp.zeros((1, 1), jnp.int32), x[:255, 255:256]], axis=0)
    for sh in (1, 2, 4, 8, 16, 32, 64, 128):
        e = e + jnp.concatenate(
            [jnp.zeros((sh, 1), jnp.int32), e[:256 - sh, :]], axis=0)
    return x + e


def _score_topk_body(en_ref, cp_ref, val_ref, idx_ref, valc_ref, idxc_ref):
    rr = jax.lax.broadcasted_iota(jnp.int32, (256, 256), 0)
    cc = jax.lax.broadcasted_iota(jnp.int32, (256, 256), 1)
    lin = rr * 256 + cc

    valc_ref[...] = jnp.zeros((2, 512, 1), jnp.float32)
    idxc_ref[...] = jnp.zeros((2, 512, 1), jnp.int32)

    ss, dd = [], []
    for b in range(2):
        s = _score_one(en_ref[b], cp_ref[b])
        bs = jax.lax.bitcast_convert_type(s, jnp.int32)
        def bisect(t, lh):
            lo, hi = lh
            mid = lo + (hi - lo) // 2
            cnt = jnp.sum((bs > mid).astype(jnp.int32))
            return jnp.where(cnt >= 300, mid, lo), jnp.where(cnt >= 300, hi, mid)
        lo, hi = jax.lax.fori_loop(
            0, 31, bisect, (jnp.int32(0), jnp.int32(0x3F800000)))
        v = hi
        gt = bs > v
        nt = 300 - jnp.sum(gt.astype(jnp.int32))
        eq = bs == v
        take_eq = eq & (_rm_cumsum(eq.astype(jnp.int32)) <= nt)
        mask = gt | take_eq
        dest = _rm_cumsum(mask.astype(jnp.int32))
        ss.append(s)
        dd.append(jnp.where(mask, dest, jnp.int32(0)))

    def extract(k, _):
        for b in range(2):
            pred = dd[b] == k + 1
            valc_ref[b, pl.ds(k, 1), 0] = jnp.sum(
                jnp.where(pred, ss[b], 0.0))[None]
            idxc_ref[b, pl.ds(k, 1), 0] = jnp.sum(
                jnp.where(pred, lin, 0))[None]
        return 0

    jax.lax.fori_loop(0, 300, extract, 0)

    riota = jax.lax.broadcasted_iota(jnp.int32, (512, 512), 1)
    for b in range(2):
        vc = valc_ref[b]
        ic = idxc_ref[b]
        vt = vc.reshape(1, 512)
        it = ic.reshape(1, 512)
        g = (vt > vc) | ((vt == vc) & (it < ic))
        rank = jnp.sum(g.astype(jnp.int32), axis=1, keepdims=True)
        oh = (rank == riota).astype(jnp.float32)
        sv = jnp.dot(vt, oh, precision=jax.lax.Precision.HIGHEST)
        si = jnp.dot(it.astype(jnp.float32), oh,
                     precision=jax.lax.Precision.HIGHEST)
        val_ref[b] = sv.reshape(512, 1)[:_K]
        idx_ref[b] = si.astype(jnp.int32).reshape(512, 1)[:_K]


def _score_topk_pass(energy, cp):
    return pl.pallas_call(
        _score_topk_body,
        grid=(1,),
        in_specs=[
            pl.BlockSpec((2, 256, 256), lambda i: (0, 0, 0)),
            pl.BlockSpec((2, 256, 256), lambda i: (0, 0, 0)),
        ],
        out_specs=[
            pl.BlockSpec((2, _K, 1), lambda i: (0, 0, 0)),
            pl.BlockSpec((2, _K, 1), lambda i: (0, 0, 0)),
        ],
        out_shape=[
            jax.ShapeDtypeStruct((2, _K, 1), jnp.float32),
            jax.ShapeDtypeStruct((2, _K, 1), jnp.int32),
        ],
        scratch_shapes=[
            pltpu.VMEM((2, 512, 1), jnp.float32),
            pltpu.VMEM((2, 512, 1), jnp.int32),
        ],
    )(energy, cp)


def kernel(enc_outputs, enc_logits):
    B, L, C = enc_outputs.shape
    eo2 = enc_outputs.reshape(B * L, C)
    el2 = enc_logits.reshape(B * L, enc_logits.shape[-1])
    energy, cp = _reduce_pass(eo2, el2, B, L)
    vals, idxs = _score_topk_pass(energy, cp)
    return (idxs[:, :300, 0], vals[:, :300, 0])


# speedup vs baseline: 1.2147x; 1.2147x over previous
import jax
import jax.numpy as jnp
from jax.experimental import pallas as pl
from jax.experimental.pallas import tpu as pltpu

_BL = 4096
_K = 304


def _reduce_body(eo_ref, el_ref, en_ref, cp_ref):
    x = eo_ref[...]
    ss = jnp.sum(x * x, axis=1, keepdims=True)
    en_ref[0] = jnp.sqrt(ss).reshape(_BL // 256, 256)
    ml = jnp.max(el_ref[...], axis=1, keepdims=True)
    cp_ref[0] = jax.nn.sigmoid(ml).reshape(_BL // 256, 256)


def _reduce_pass(eo2, el2, B, L):
    n = B * L
    nb = n // _BL
    rows = _BL // 256
    bpb = L // _BL
    return pl.pallas_call(
        _reduce_body,
        grid=(nb,),
        in_specs=[
            pl.BlockSpec((_BL, 256), lambda i: (i, 0)),
            pl.BlockSpec((_BL, 80), lambda i: (i, 0)),
        ],
        out_specs=[
            pl.BlockSpec((1, rows, 256), lambda i: (i // bpb, i % bpb, 0)),
            pl.BlockSpec((1, rows, 256), lambda i: (i // bpb, i % bpb, 0)),
        ],
        out_shape=[
            jax.ShapeDtypeStruct((B, 256, 256), jnp.float32),
            jax.ShapeDtypeStruct((B, 256, 256), jnp.float32),
        ],
    )(eo2, el2)


def _score_one(e, cp):
    zr = jnp.zeros((1, 256), jnp.float32)
    zc = jnp.zeros((256, 1), jnp.float32)

    def sh(a, dh, dw):
        if dh == 1:
            a = jnp.concatenate([a[1:, :], zr], axis=0)
        elif dh == -1:
            a = jnp.concatenate([zr, a[:-1, :]], axis=0)
        if dw == 1:
            a = jnp.concatenate([a[:, 1:], zc], axis=1)
        elif dw == -1:
            a = jnp.concatenate([zc, a[:, :-1]], axis=1)
        return a

    win = sh(e, -1, -1)
    for dh, dw in ((-1, 0), (-1, 1), (0, -1), (0, 0), (0, 1),
                   (1, -1), (1, 0), (1, 1)):
        win = win + sh(e, dh, dw)
    dens = win * jnp.float32(1.0 / 9.0)
    mn = jnp.min(dens)
    mx = jnp.max(dens)
    denom = (mx - mn) + 1e-06
    return cp * (1.0 - 0.4) + ((dens - mn) / denom) * 0.4


def _rm_cumsum(m):
    x = m
    for sh in (1, 2, 4, 8, 16, 32, 64, 128):
        x = x + jnp.concatenate(
            [jnp.zeros((256, sh), jnp.int32), x[:, :256 - sh]], axis=1)
    e = jnp.concatenate(
        [jnp.zeros((1, 1), jnp.int32), x[:255, 255:256]], axis=0)
    for sh in (1, 2, 4, 8, 16, 32, 64, 128):
        e = e + jnp.concatenate(
            [jnp.zeros((sh, 1), jnp.int32), e[:256 - sh, :]], axis=0)
    return x + e


def _score_topk_body(en_ref, cp_ref, val_ref, idx_ref, valc_ref, idxc_ref):
    rr = jax.lax.broadcasted_iota(jnp.int32, (256, 256), 0)
    cc = jax.lax.broadcasted_iota(jnp.int32, (256, 256), 1)
    lin = rr * 256 + cc

    valc_ref[...] = jnp.zeros((2, 512, 1), jnp.float32)
    idxc_ref[...] = jnp.zeros((2, 512, 1), jnp.int32)

    ss = [_score_one(en_ref[b], cp_ref[b]) for b in range(2)]
    bb = [jax.lax.bitcast_convert_type(s, jnp.int32) for s in ss]

    def bisect(t, lh):
        out = []
        for b in range(2):
            lo, hi = lh[2 * b], lh[2 * b + 1]
            mid = lo + (hi - lo) // 2
            cnt = jnp.sum((bb[b] > mid).astype(jnp.int32))
            out += [jnp.where(cnt >= 300, mid, lo),
                    jnp.where(cnt >= 300, hi, mid)]
        return tuple(out)

    z, one = jnp.int32(0), jnp.int32(0x3F800000)
    lh = jax.lax.fori_loop(0, 31, bisect, (z, one, z, one))

    dd = []
    for b in range(2):
        v = lh[2 * b + 1]
        gt = bb[b] > v
        nt = 300 - jnp.sum(gt.astype(jnp.int32))
        eq = bb[b] == v
        take_eq = eq & (_rm_cumsum(eq.astype(jnp.int32)) <= nt)
        mask = gt | take_eq
        dest = _rm_cumsum(mask.astype(jnp.int32))
        dd.append(jnp.where(mask, dest, jnp.int32(0)))

    def extract(t, _):
        for u in range(4):
            k = t * 4 + u
            for b in range(2):
                pred = dd[b] == k + 1
                valc_ref[b, pl.ds(k, 1), 0] = jnp.sum(
                    jnp.where(pred, ss[b], 0.0))[None]
                idxc_ref[b, pl.ds(k, 1), 0] = jnp.sum(
                    jnp.where(pred, lin, 0))[None]
        return 0

    jax.lax.fori_loop(0, 75, extract, 0)

    riota = jax.lax.broadcasted_iota(jnp.int32, (512, 512), 1)
    for b in range(2):
        vc = valc_ref[b]
        ic = idxc_ref[b]
        vt = vc.reshape(1, 512)
        it = ic.reshape(1, 512)
        g = (vt > vc) | ((vt == vc) & (it < ic))
        rank = jnp.sum(g.astype(jnp.int32), axis=1, keepdims=True)
        oh = (rank == riota).astype(jnp.float32)
        sv = jnp.dot(vt, oh, precision=jax.lax.Precision.HIGHEST)
        si = jnp.dot(it.astype(jnp.float32), oh,
                     precision=jax.lax.Precision.HIGHEST)
        val_ref[b] = sv.reshape(512, 1)[:_K]
        idx_ref[b] = si.astype(jnp.int32).reshape(512, 1)[:_K]


def _score_topk_pass(energy, cp):
    return pl.pallas_call(
        _score_topk_body,
        grid=(1,),
        in_specs=[
            pl.BlockSpec((2, 256, 256), lambda i: (0, 0, 0)),
            pl.BlockSpec((2, 256, 256), lambda i: (0, 0, 0)),
        ],
        out_specs=[
            pl.BlockSpec((2, _K, 1), lambda i: (0, 0, 0)),
            pl.BlockSpec((2, _K, 1), lambda i: (0, 0, 0)),
        ],
        out_shape=[
            jax.ShapeDtypeStruct((2, _K, 1), jnp.float32),
            jax.ShapeDtypeStruct((2, _K, 1), jnp.int32),
        ],
        scratch_shapes=[
            pltpu.VMEM((2, 512, 1), jnp.float32),
            pltpu.VMEM((2, 512, 1), jnp.int32),
        ],
    )(energy, cp)


def kernel(enc_outputs, enc_logits):
    B, L, C = enc_outputs.shape
    eo2 = enc_outputs.reshape(B * L, C)
    el2 = enc_logits.reshape(B * L, enc_logits.shape[-1])
    energy, cp = _reduce_pass(eo2, el2, B, L)
    vals, idxs = _score_topk_pass(energy, cp)
    return (idxs[:, :300, 0], vals[:, :300, 0])


# speedup vs baseline: 1.2535x; 1.0320x over previous
import jax
import jax.numpy as jnp
from jax.experimental import pallas as pl
from jax.experimental.pallas import tpu as pltpu

_BL = 8192
_K = 304


def _reduce_body(eo_ref, el_ref, en_ref, cp_ref):
    x = eo_ref[...]
    ss = jnp.sum(x * x, axis=1, keepdims=True)
    en_ref[0] = jnp.sqrt(ss).reshape(_BL // 256, 256)
    ml = jnp.max(el_ref[...], axis=1, keepdims=True)
    cp_ref[0] = jax.nn.sigmoid(ml).reshape(_BL // 256, 256)


def _reduce_pass(eo2, el2, B, L):
    n = B * L
    nb = n // _BL
    rows = _BL // 256
    bpb = L // _BL
    return pl.pallas_call(
        _reduce_body,
        grid=(nb,),
        in_specs=[
            pl.BlockSpec((_BL, 256), lambda i: (i, 0)),
            pl.BlockSpec((_BL, 80), lambda i: (i, 0)),
        ],
        out_specs=[
            pl.BlockSpec((1, rows, 256), lambda i: (i // bpb, i % bpb, 0)),
            pl.BlockSpec((1, rows, 256), lambda i: (i // bpb, i % bpb, 0)),
        ],
        out_shape=[
            jax.ShapeDtypeStruct((B, 256, 256), jnp.float32),
            jax.ShapeDtypeStruct((B, 256, 256), jnp.float32),
        ],
    )(eo2, el2)


def _score_one(e, cp):
    zr = jnp.zeros((1, 256), jnp.float32)
    zc = jnp.zeros((256, 1), jnp.float32)

    def sh(a, dh, dw):
        if dh == 1:
            a = jnp.concatenate([a[1:, :], zr], axis=0)
        elif dh == -1:
            a = jnp.concatenate([zr, a[:-1, :]], axis=0)
        if dw == 1:
            a = jnp.concatenate([a[:, 1:], zc], axis=1)
        elif dw == -1:
            a = jnp.concatenate([zc, a[:, :-1]], axis=1)
        return a

    win = sh(e, -1, -1)
    for dh, dw in ((-1, 0), (-1, 1), (0, -1), (0, 0), (0, 1),
                   (1, -1), (1, 0), (1, 1)):
        win = win + sh(e, dh, dw)
    dens = win * jnp.float32(1.0 / 9.0)
    mn = jnp.min(dens)
    mx = jnp.max(dens)
    denom = (mx - mn) + 1e-06
    return cp * (1.0 - 0.4) + ((dens - mn) / denom) * 0.4


def _rm_cumsum(m):
    x = m
    for sh in (1, 2, 4, 8, 16, 32, 64, 128):
        x = x + jnp.concatenate(
            [jnp.zeros((256, sh), jnp.int32), x[:, :256 - sh]], axis=1)
    e = jnp.concatenate(
        [jnp.zeros((1, 1), jnp.int32), x[:255, 255:256]], axis=0)
    for sh in (1, 2, 4, 8, 16, 32, 64, 128):
        e = e + jnp.concatenate(
            [jnp.zeros((sh, 1), jnp.int32), e[:256 - sh, :]], axis=0)
    return x + e


def _score_topk_body(en_ref, cp_ref, val_ref, idx_ref, valc_ref, idxc_ref):
    rr = jax.lax.broadcasted_iota(jnp.int32, (256, 256), 0)
    cc = jax.lax.broadcasted_iota(jnp.int32, (256, 256), 1)
    lin = rr * 256 + cc

    valc_ref[...] = jnp.zeros((2, 512, 1), jnp.float32)
    idxc_ref[...] = jnp.zeros((2, 512, 1), jnp.int32)

    ss = [_score_one(en_ref[b], cp_ref[b]) for b in range(2)]
    bb = [jax.lax.bitcast_convert_type(s, jnp.int32) for s in ss]

    def bisect(t, lh):
        out = []
        for b in range(2):
            lo, hi = lh[2 * b], lh[2 * b + 1]
            mid = lo + (hi - lo) // 2
            cnt = jnp.sum((bb[b] > mid).astype(jnp.int32))
            out += [jnp.where(cnt >= 300, mid, lo),
                    jnp.where(cnt >= 300, hi, mid)]
        return tuple(out)

    z, one = jnp.int32(0), jnp.int32(0x3F800000)
    lh = jax.lax.fori_loop(0, 31, bisect, (z, one, z, one))

    dd = []
    for b in range(2):
        v = lh[2 * b + 1]
        gt = bb[b] > v
        nt = 300 - jnp.sum(gt.astype(jnp.int32))
        eq = bb[b] == v
        take_eq = eq & (_rm_cumsum(eq.astype(jnp.int32)) <= nt)
        mask = gt | take_eq
        dest = _rm_cumsum(mask.astype(jnp.int32))
        dd.append(jnp.where(mask, dest, jnp.int32(0)))

    def extract(t, _):
        for u in range(4):
            k = t * 4 + u
            for b in range(2):
                pred = dd[b] == k + 1
                valc_ref[b, pl.ds(k, 1), 0] = jnp.sum(
                    jnp.where(pred, ss[b], 0.0))[None]
                idxc_ref[b, pl.ds(k, 1), 0] = jnp.sum(
                    jnp.where(pred, lin, 0))[None]
        return 0

    jax.lax.fori_loop(0, 75, extract, 0)

    riota = jax.lax.broadcasted_iota(jnp.int32, (512, 512), 1)
    for b in range(2):
        vc = valc_ref[b]
        ic = idxc_ref[b]
        vt = vc.reshape(1, 512)
        it = ic.reshape(1, 512)
        g = (vt > vc) | ((vt == vc) & (it < ic))
        rank = jnp.sum(g.astype(jnp.int32), axis=1, keepdims=True)
        oh = (rank == riota).astype(jnp.float32)
        sv = jnp.dot(vt, oh, precision=jax.lax.Precision.HIGHEST)
        si = jnp.dot(it.astype(jnp.float32), oh,
                     precision=jax.lax.Precision.HIGHEST)
        val_ref[b] = sv.reshape(512, 1)[:_K]
        idx_ref[b] = si.astype(jnp.int32).reshape(512, 1)[:_K]


def _score_topk_pass(energy, cp):
    return pl.pallas_call(
        _score_topk_body,
        grid=(1,),
        in_specs=[
            pl.BlockSpec((2, 256, 256), lambda i: (0, 0, 0)),
            pl.BlockSpec((2, 256, 256), lambda i: (0, 0, 0)),
        ],
        out_specs=[
            pl.BlockSpec((2, _K, 1), lambda i: (0, 0, 0)),
            pl.BlockSpec((2, _K, 1), lambda i: (0, 0, 0)),
        ],
        out_shape=[
            jax.ShapeDtypeStruct((2, _K, 1), jnp.float32),
            jax.ShapeDtypeStruct((2, _K, 1), jnp.int32),
        ],
        scratch_shapes=[
            pltpu.VMEM((2, 512, 1), jnp.float32),
            pltpu.VMEM((2, 512, 1), jnp.int32),
        ],
    )(energy, cp)


def kernel(enc_outputs, enc_logits):
    B, L, C = enc_outputs.shape
    eo2 = enc_outputs.reshape(B * L, C)
    el2 = enc_logits.reshape(B * L, enc_logits.shape[-1])
    energy, cp = _reduce_pass(eo2, el2, B, L)
    vals, idxs = _score_topk_pass(energy, cp)
    return (idxs[:, :300, 0], vals[:, :300, 0])
